# baseline (speedup 1.0000x reference)
import functools

import jax
import jax.numpy as jnp
import numpy as np
from jax import lax
from jax.experimental import pallas as pl
from jax.experimental.pallas import tpu as pltpu
from jax.experimental.pallas import tpu_sc as plsc

_N = 10000
_E = 320000
_H = 128
_NL = 4
_G = 64

_NC = 2
_NS = 16
_NW = _NC * _NS
_CHUNK = 80
_CA = 48
_CB = 32
_EPT = _E // _NW
_NCHUNK = _EPT // _CHUNK
_PKROWS = 64
_STRIPE = 640
_BLK = 80

_w = np.arange(64)
_NAT128 = np.concatenate([32 * (_w // 16) + (_w % 16),
                          32 * (_w // 16) + 16 + (_w % 16)]).astype(np.int32)
_PERM512 = np.concatenate([_NAT128 + 128 * b for b in range(4)])
_PERM256 = np.concatenate([_NAT128, _NAT128 + 128])

_LOG1P_C = (
    0.9996204346781663, -0.4866434251928923, 0.25462270211492277,
    -0.0747363009421317,
)


def _log1p01(t):
    p = jnp.full((16,), _LOG1P_C[-1], dtype=jnp.float32)
    for c in _LOG1P_C[-2::-1]:
        p = p * t + c
    return t * p


def _sigmoid(x):
    return 1.0 / (1.0 + jnp.exp(-x))


def _softplus(x):
    return jnp.maximum(x, 0.0) + _log1p01(jnp.exp(-jnp.abs(x)))


def _lo_f32(v):
    return lax.bitcast_convert_type(lax.shift_left(v, 16), jnp.float32)


def _hi_f32(v):
    return lax.bitcast_convert_type(jnp.bitwise_and(v, jnp.int32(-65536)),
                                    jnp.float32)


def _edge_body(layer, pk_hbm, pd_hbm, ps_hbm, eb_hbm, out_hbm,
               pk_t, sa_a, da_a, sa_b, da_b, d_sc,
               ad, as_, ae, bd, bs, be, msg_v, agg_sh, sema, semb):
    ecol = _H * layer
    cid = lax.axis_index("c")
    sid = lax.axis_index("s")
    wid = cid * _NS + sid

    zeros16 = jnp.zeros((16,), jnp.float32)

    def _zero_row(i, carry):
        for j in range(8):
            msg_v[i, pl.ds(16 * j, 16)] = zeros16
        return carry

    lax.fori_loop(0, _BLK, _zero_row, 0)
    nblk = jnp.where(sid == _NS - 1, (_N - (_NS - 1) * _STRIPE) // _BLK,
                     _STRIPE // _BLK)

    def _zero_blk(b, carry):
        pltpu.sync_copy(msg_v, agg_sh.at[pl.ds(sid * _STRIPE + b * _BLK, _BLK)])
        return carry

    lax.fori_loop(0, nblk, _zero_blk, 0)

    pltpu.sync_copy(pk_hbm.at[wid, pl.ds(0, _PKROWS)], pk_t)
    plsc.subcore_barrier()

    def _unpack(row, g0, g1, s_ref, d_ref):
        for g in range(g0, g1):
            v = pk_t[row, pl.ds(16 * g, 16)]
            s_ref[pl.ds(16 * (g - g0), 16)] = jnp.bitwise_and(
                v, jnp.int32(0x3FFF))
            d_ref[pl.ds(16 * (g - g0), 16)] = jnp.bitwise_and(
                lax.shift_right_logical(v, 14), jnp.int32(0x3FFF))

    def _act(gd, gs, ge, n, moff):
        def _body(i):
            for j in range(4):
                w = 16 * j
                vfd = gd[i, pl.ds(w, 16)]
                vfs = gs[i, pl.ds(w, 16)]
                vfe = ge[i, pl.ds(w, 16)]
                vsd = gd[i, pl.ds(64 + w, 16)]
                vss = gs[i, pl.ds(64 + w, 16)]
                vse = ge[i, pl.ds(64 + w, 16)]
                uf_lo = _lo_f32(vfd) + _lo_f32(vfs) + _lo_f32(vfe)
                uf_hi = _hi_f32(vfd) + _hi_f32(vfs) + _hi_f32(vfe)
                us_lo = _lo_f32(vsd) + _lo_f32(vss) + _lo_f32(vse)
                us_hi = _hi_f32(vsd) + _hi_f32(vss) + _hi_f32(vse)
                msg_v[moff + i, pl.ds(32 * j, 16)] = (
                    _sigmoid(uf_lo) * _softplus(us_lo))
                msg_v[moff + i, pl.ds(32 * j + 16, 16)] = (
                    _sigmoid(uf_hi) * _softplus(us_hi))

        plsc.parallel_loop(0, n, unroll=2)(_body)

    def _ga(q):
        base = wid * _EPT + q * _CHUNK
        pltpu.async_copy(pd_hbm.at[da_a], ad, sema)
        pltpu.async_copy(ps_hbm.at[sa_a], as_, sema)
        pltpu.async_copy(eb_hbm.at[pl.ds(base, _CA), pl.ds(ecol, _H)], ae,
                         sema)

    def _gb(q):
        base = wid * _EPT + q * _CHUNK
        pltpu.async_copy(pd_hbm.at[da_b], bd, semb)
        pltpu.async_copy(ps_hbm.at[sa_b], bs, semb)
        pltpu.async_copy(eb_hbm.at[pl.ds(base + _CA, _CB), pl.ds(ecol, _H)],
                         be, semb)

    def _wait3(idx_ref, ref, sem):
        for _ in range(3):
            pltpu.make_async_copy(pd_hbm.at[idx_ref], ref, sem).wait()

    def _copy_idx(src_ref, dst_off, ngroups):
        for g in range(ngroups):
            d_sc[pl.ds(dst_off + 16 * g, 16)] = src_ref[pl.ds(16 * g, 16)]

    _unpack(0, 0, 3, sa_a, da_a)
    _unpack(0, 3, 5, sa_b, da_b)
    _ga(0)

    def _chunk(q, carry):
        _gb(q)

        @pl.when(q == _PKROWS - 1)
        def _():
            pltpu.sync_copy(pk_hbm.at[wid, pl.ds(_PKROWS, _NCHUNK - _PKROWS)],
                            pk_t.at[pl.ds(0, _NCHUNK - _PKROWS)])

        _wait3(da_a, ad, sema)
        _act(ad, as_, ae, _CA, 0)
        _copy_idx(da_a, 0, _CA // 16)
        row_n = jnp.where(q + 1 >= _PKROWS, q + 1 - _PKROWS, q + 1)

        @pl.when(q + 1 < _NCHUNK)
        def _():
            _unpack(row_n, 0, 3, sa_a, da_a)
            _ga(q + 1)

        _wait3(da_b, bd, semb)
        _act(bd, bs, be, _CB, _CA)
        _copy_idx(da_b, _CA, _CB // 16)
        pltpu.sync_copy(msg_v, agg_sh.at[d_sc], add=True)

        @pl.when(q + 1 < _NCHUNK)
        def _():
            _unpack(row_n, 3, 5, sa_b, da_b)

        return carry

    lax.fori_loop(0, _NCHUNK, _chunk, 0)
    plsc.subcore_barrier()

    def _out_blk(b, carry):
        r0 = sid * _STRIPE + b * _BLK
        pltpu.sync_copy(agg_sh.at[pl.ds(r0, _BLK)], msg_v)
        pltpu.sync_copy(msg_v, out_hbm.at[cid, pl.ds(r0, _BLK)])
        return carry

    lax.fori_loop(0, nblk, _out_blk, 0)


def _make_edge_pass(layer):
  return pl.kernel(
    functools.partial(_edge_body, layer),
    out_type=jax.ShapeDtypeStruct((_NC, _N, _H), jnp.float32),
    mesh=plsc.VectorSubcoreMesh(core_axis_name="c", subcore_axis_name="s"),
    scratch_types=[
        pltpu.VMEM((_PKROWS, _CHUNK), jnp.int32),
        pltpu.VMEM((_CA,), jnp.int32),
        pltpu.VMEM((_CA,), jnp.int32),
        pltpu.VMEM((_CB,), jnp.int32),
        pltpu.VMEM((_CB,), jnp.int32),
        pltpu.VMEM((_CHUNK,), jnp.int32),
        pltpu.VMEM((_CA, _H), jnp.int32),
        pltpu.VMEM((_CA, _H), jnp.int32),
        pltpu.VMEM((_CA, _H), jnp.int32),
        pltpu.VMEM((_CB, _H), jnp.int32),
        pltpu.VMEM((_CB, _H), jnp.int32),
        pltpu.VMEM((_CB, _H), jnp.int32),
        pltpu.VMEM((_CHUNK, _H), jnp.float32),
        pltpu.VMEM_SHARED((_N, _H), jnp.float32),
        pltpu.SemaphoreType.DMA,
        pltpu.SemaphoreType.DMA,
    ],
  )


_EDGE_PASS = [_make_edge_pass(l) for l in range(_NL)]


_BM = 1000
_BME = 4000

_P_OUT_SPECS = [
    pl.BlockSpec((_BM, _H), lambda i: (i, 0)),
    pl.BlockSpec((_BM, _H), lambda i: (i, 0)),
    pl.BlockSpec((_BM, _H), lambda i: (i, 0)),
]
_P_OUT_SHAPE = [
    jax.ShapeDtypeStruct((_N, _H), jnp.float32),
    jax.ShapeDtypeStruct((_N, _H), jnp.int32),
    jax.ShapeDtypeStruct((_N, _H), jnp.int32),
]


def _pack16(a, b):
    au = lax.bitcast_convert_type(a.astype(jnp.bfloat16),
                                  jnp.uint16).astype(jnp.int32)
    bu = lax.bitcast_convert_type(b.astype(jnp.bfloat16),
                                  jnp.uint16).astype(jnp.int32)
    return jnp.bitwise_or(au, lax.shift_left(bu, 16))


def _pack_role(p, c0):
    return jnp.concatenate(
        [_pack16(p[:, c0:c0 + 64], p[:, c0 + 64:c0 + 128]),
         _pack16(p[:, c0 + 128:c0 + 192], p[:, c0 + 192:c0 + 256])], axis=1)


def _split_proj(h, w512, h_ref, pd_ref, ps_ref):
    h_ref[...] = h
    p = jnp.dot(h, w512, preferred_element_type=jnp.float32)
    pd_ref[...] = _pack_role(p, 0)
    ps_ref[...] = _pack_role(p, 256)


def _stage0_body(atom_ref, wemb_ref, bemb_ref, w512_ref,
                 h_ref, pd_ref, ps_ref):
    a = atom_ref[...]
    h = jnp.dot(a, wemb_ref[...], preferred_element_type=jnp.float32) + bemb_ref[...]
    h = jnp.where(h > 0, h, jnp.exp(h) - 1.0)
    _split_proj(h, w512_ref[...], h_ref, pd_ref, ps_ref)


def _stage0(atom_feat, W_embed, b_embed, W512_0):
    return pl.pallas_call(
        _stage0_body,
        grid=(_N // _BM,),
        in_specs=[
            pl.BlockSpec((_BM, _H), lambda i: (i, 0)),
            pl.BlockSpec((_H, _H), lambda i: (0, 0)),
            pl.BlockSpec((1, _H), lambda i: (0, 0)),
            pl.BlockSpec((_H, 4 * _H), lambda i: (0, 0)),
        ],
        out_specs=_P_OUT_SPECS,
        out_shape=_P_OUT_SHAPE,
    )(atom_feat, W_embed, b_embed, W512_0)


def _stageL_body(h_ref, agg_ref, w512_ref, h_out_ref, pd_ref, ps_ref):
    h = h_ref[...] + agg_ref[0] + agg_ref[1]
    _split_proj(h, w512_ref[...], h_out_ref, pd_ref, ps_ref)


def _stageL(h, aggp, W512_l):
    return pl.pallas_call(
        _stageL_body,
        grid=(_N // _BM,),
        in_specs=[
            pl.BlockSpec((_BM, _H), lambda i: (i, 0)),
            pl.BlockSpec((_NC, _BM, _H), lambda i: (0, i, 0)),
            pl.BlockSpec((_H, 4 * _H), lambda i: (0, 0)),
        ],
        out_specs=_P_OUT_SPECS,
        out_shape=_P_OUT_SHAPE,
    )(h, aggp, W512_l)


def _bond_body(bfeat_ref, w_ref, b_ref, out_ref):
    p = (jnp.dot(bfeat_ref[...], w_ref[...],
                 preferred_element_type=jnp.float32) + b_ref[...])
    out_ref[...] = jnp.concatenate(
        [_pack16(p[:, :64], p[:, 64:128]),
         _pack16(p[:, 128:192], p[:, 192:])], axis=1)


def _bond(bond_feat, Wbond_l, bias_l):
    return pl.pallas_call(
        _bond_body,
        grid=(_E // _BME,),
        in_specs=[
            pl.BlockSpec((_BME, 16), lambda i: (i, 0)),
            pl.BlockSpec((16, 2 * _H), lambda i: (0, 0)),
            pl.BlockSpec((1, 2 * _H), lambda i: (0, 0)),
        ],
        out_specs=pl.BlockSpec((_BME, _H), lambda i: (i, 0)),
        out_shape=jax.ShapeDtypeStruct((_E, _H), jnp.int32),
    )(bond_feat, Wbond_l, bias_l)


def _pool_body(h_ref, agg_ref, gidx_ref, wout_ref, bout_ref, out_ref):
    h = h_ref[...] + agg_ref[0] + agg_ref[1]
    g = gidx_ref[...]
    iota = lax.broadcasted_iota(jnp.int32, (_G, _N), 0)
    onehot = (iota == g).astype(jnp.float32)
    sums = jnp.dot(onehot, h, preferred_element_type=jnp.float32)
    counts = jnp.sum(onehot, axis=1, keepdims=True)
    pooled = sums / jnp.maximum(counts, 1.0)
    out_ref[...] = (jnp.dot(pooled, wout_ref[...],
                            preferred_element_type=jnp.float32) + bout_ref[...])


def _pool(h, aggp, gidx, W_out, b_out):
    return pl.pallas_call(
        _pool_body,
        out_shape=jax.ShapeDtypeStruct((_G, _H), jnp.float32),
    )(h, aggp, gidx, W_out, b_out)


def kernel(atom_feat, bond_idx, graph_idx, bond_feat, W_embed, b_embed,
           Wf, bf, Ws, bs, W_out, b_out):
    src = bond_idx[0].astype(jnp.int32)
    dst = bond_idx[1].astype(jnp.int32)
    pk = (src | (dst << 14)).reshape(_NW, _NCHUNK, _CHUNK)
    gidx = graph_idx.astype(jnp.int32).reshape(1, _N)

    W512 = jnp.concatenate(
        [Wf[:, :_H, :], Ws[:, :_H, :], Wf[:, _H:2 * _H, :], Ws[:, _H:2 * _H, :]],
        axis=2)[:, :, _PERM512]
    Wbond = jnp.concatenate([Wf[:, 2 * _H:, :], Ws[:, 2 * _H:, :]],
                            axis=2)[:, :, _PERM256]
    bias = jnp.concatenate([bf, bs], axis=1)[:, _PERM256].reshape(_NL, 1, 2 * _H)

    h, pd, ps = _stage0(atom_feat, W_embed, b_embed.reshape(1, _H), W512[0])
    aggp = None
    for l in range(_NL):
        eb = _bond(bond_feat, Wbond[l], bias[l])
        aggp = _EDGE_PASS[0](pk, pd, ps, eb)
        if l + 1 < _NL:
            h, pd, ps = _stageL(h, aggp, W512[l + 1])
    return _pool(h, aggp, gidx, W_out, b_out.reshape(1, _H))


# speedup vs baseline: 1.0209x; 1.0209x over previous
import functools

import jax
import jax.numpy as jnp
import numpy as np
from jax import lax
from jax.experimental import pallas as pl
from jax.experimental.pallas import tpu as pltpu
from jax.experimental.pallas import tpu_sc as plsc

_N = 10000
_E = 320000
_H = 128
_NL = 4
_G = 64

_NC = 2
_NS = 16
_NW = _NC * _NS
_CHUNK = 80
_CA = 48
_CB = 32
_EPT = _E // _NW
_NCHUNK = _EPT // _CHUNK
_PKROWS = 64
_STRIPE = 640
_BLK = 80

_w = np.arange(64)
_NAT128 = np.concatenate([32 * (_w // 16) + (_w % 16),
                          32 * (_w // 16) + 16 + (_w % 16)]).astype(np.int32)
_PERM512 = np.concatenate([_NAT128 + 128 * b for b in range(4)])
_PERM256 = np.concatenate([_NAT128, _NAT128 + 128])

_LOG1P_C = (
    0.9996204346781663, -0.4866434251928923, 0.25462270211492277,
    -0.0747363009421317,
)


def _log1p01(t):
    p = jnp.full((16,), _LOG1P_C[-1], dtype=jnp.float32)
    for c in _LOG1P_C[-2::-1]:
        p = p * t + c
    return t * p


def _sigmoid(x):
    return 1.0 / (1.0 + jnp.exp(-x))


def _softplus(x):
    return jnp.maximum(x, 0.0) + _log1p01(jnp.exp(-jnp.abs(x)))


def _lo_f32(v):
    return lax.bitcast_convert_type(lax.shift_left(v, 16), jnp.float32)


def _hi_f32(v):
    return lax.bitcast_convert_type(jnp.bitwise_and(v, jnp.int32(-65536)),
                                    jnp.float32)


def _edge_body(layer, pk_hbm, pd_hbm, ps_hbm, eb_hbm, out_hbm,
               pk_t, sa_a, da_a, sa_b, da_b, d_sc,
               ad, as_, ae, bd, bs, be, msg_v, agg_sh, sema, semb):
    ecol = _H * layer
    cid = lax.axis_index("c")
    sid = lax.axis_index("s")
    wid = cid * _NS + sid

    zeros16 = jnp.zeros((16,), jnp.float32)

    def _zero_row(i, carry):
        for j in range(8):
            msg_v[i, pl.ds(16 * j, 16)] = zeros16
        return carry

    lax.fori_loop(0, _BLK, _zero_row, 0)
    nblk = jnp.where(sid == _NS - 1, (_N - (_NS - 1) * _STRIPE) // _BLK,
                     _STRIPE // _BLK)

    def _zero_blk(b, carry):
        pltpu.sync_copy(msg_v, agg_sh.at[pl.ds(sid * _STRIPE + b * _BLK, _BLK)])
        return carry

    lax.fori_loop(0, nblk, _zero_blk, 0)

    pltpu.sync_copy(pk_hbm.at[wid, pl.ds(0, _PKROWS)], pk_t)
    plsc.subcore_barrier()

    def _unpack(row, g0, g1, s_ref, d_ref):
        for g in range(g0, g1):
            v = pk_t[row, pl.ds(16 * g, 16)]
            s_ref[pl.ds(16 * (g - g0), 16)] = jnp.bitwise_and(
                v, jnp.int32(0x3FFF))
            d_ref[pl.ds(16 * (g - g0), 16)] = jnp.bitwise_and(
                lax.shift_right_logical(v, 14), jnp.int32(0x3FFF))

    def _act(gd, gs, ge, n, moff):
        def _body(i, c2):
            for j in range(4):
                w = 16 * j
                vfd = gd[i, pl.ds(w, 16)]
                vfs = gs[i, pl.ds(w, 16)]
                vfe = ge[i, pl.ds(w, 16)]
                vsd = gd[i, pl.ds(64 + w, 16)]
                vss = gs[i, pl.ds(64 + w, 16)]
                vse = ge[i, pl.ds(64 + w, 16)]
                uf_lo = _lo_f32(vfd) + _lo_f32(vfs) + _lo_f32(vfe)
                uf_hi = _hi_f32(vfd) + _hi_f32(vfs) + _hi_f32(vfe)
                us_lo = _lo_f32(vsd) + _lo_f32(vss) + _lo_f32(vse)
                us_hi = _hi_f32(vsd) + _hi_f32(vss) + _hi_f32(vse)
                msg_v[moff + i, pl.ds(32 * j, 16)] = (
                    _sigmoid(uf_lo) * _softplus(us_lo))
                msg_v[moff + i, pl.ds(32 * j + 16, 16)] = (
                    _sigmoid(uf_hi) * _softplus(us_hi))
            return c2

        lax.fori_loop(0, n, _body, 0)

    def _ga(q):
        base = wid * _EPT + q * _CHUNK
        pltpu.async_copy(pd_hbm.at[da_a], ad, sema)
        pltpu.async_copy(ps_hbm.at[sa_a], as_, sema)
        pltpu.async_copy(eb_hbm.at[pl.ds(base, _CA), pl.ds(ecol, _H)], ae,
                         sema)

    def _gb(q):
        base = wid * _EPT + q * _CHUNK
        pltpu.async_copy(pd_hbm.at[da_b], bd, semb)
        pltpu.async_copy(ps_hbm.at[sa_b], bs, semb)
        pltpu.async_copy(eb_hbm.at[pl.ds(base + _CA, _CB), pl.ds(ecol, _H)],
                         be, semb)

    def _wait3(idx_ref, ref, sem):
        for _ in range(3):
            pltpu.make_async_copy(pd_hbm.at[idx_ref], ref, sem).wait()

    def _copy_idx(src_ref, dst_off, ngroups):
        for g in range(ngroups):
            d_sc[pl.ds(dst_off + 16 * g, 16)] = src_ref[pl.ds(16 * g, 16)]

    _unpack(0, 0, 3, sa_a, da_a)
    _unpack(0, 3, 5, sa_b, da_b)
    _ga(0)

    def _chunk(q, carry):
        _gb(q)

        @pl.when(q == _PKROWS - 1)
        def _():
            pltpu.sync_copy(pk_hbm.at[wid, pl.ds(_PKROWS, _NCHUNK - _PKROWS)],
                            pk_t.at[pl.ds(0, _NCHUNK - _PKROWS)])

        _wait3(da_a, ad, sema)
        _act(ad, as_, ae, _CA, 0)
        _copy_idx(da_a, 0, _CA // 16)
        row_n = jnp.where(q + 1 >= _PKROWS, q + 1 - _PKROWS, q + 1)

        @pl.when(q + 1 < _NCHUNK)
        def _():
            _unpack(row_n, 0, 3, sa_a, da_a)
            _ga(q + 1)

        _wait3(da_b, bd, semb)
        _act(bd, bs, be, _CB, _CA)
        _copy_idx(da_b, _CA, _CB // 16)
        pltpu.sync_copy(msg_v, agg_sh.at[d_sc], add=True)

        @pl.when(q + 1 < _NCHUNK)
        def _():
            _unpack(row_n, 3, 5, sa_b, da_b)

        return carry

    lax.fori_loop(0, _NCHUNK, _chunk, 0)
    plsc.subcore_barrier()

    def _out_blk(b, carry):
        r0 = sid * _STRIPE + b * _BLK
        pltpu.sync_copy(agg_sh.at[pl.ds(r0, _BLK)], msg_v)
        pltpu.sync_copy(msg_v, out_hbm.at[cid, pl.ds(r0, _BLK)])
        return carry

    lax.fori_loop(0, nblk, _out_blk, 0)


def _make_edge_pass(layer):
  return pl.kernel(
    functools.partial(_edge_body, layer),
    out_type=jax.ShapeDtypeStruct((_NC, _N, _H), jnp.float32),
    mesh=plsc.VectorSubcoreMesh(core_axis_name="c", subcore_axis_name="s"),
    scratch_types=[
        pltpu.VMEM((_PKROWS, _CHUNK), jnp.int32),
        pltpu.VMEM((_CA,), jnp.int32),
        pltpu.VMEM((_CA,), jnp.int32),
        pltpu.VMEM((_CB,), jnp.int32),
        pltpu.VMEM((_CB,), jnp.int32),
        pltpu.VMEM((_CHUNK,), jnp.int32),
        pltpu.VMEM((_CA, _H), jnp.int32),
        pltpu.VMEM((_CA, _H), jnp.int32),
        pltpu.VMEM((_CA, _H), jnp.int32),
        pltpu.VMEM((_CB, _H), jnp.int32),
        pltpu.VMEM((_CB, _H), jnp.int32),
        pltpu.VMEM((_CB, _H), jnp.int32),
        pltpu.VMEM((_CHUNK, _H), jnp.float32),
        pltpu.VMEM_SHARED((_N, _H), jnp.float32),
        pltpu.SemaphoreType.DMA,
        pltpu.SemaphoreType.DMA,
    ],
  )


_EDGE_PASS = [_make_edge_pass(l) for l in range(_NL)]


_BM = 1000
_BME = 4000

_P_OUT_SPECS = [
    pl.BlockSpec((_BM, _H), lambda i: (i, 0)),
    pl.BlockSpec((_BM, _H), lambda i: (i, 0)),
    pl.BlockSpec((_BM, _H), lambda i: (i, 0)),
]
_P_OUT_SHAPE = [
    jax.ShapeDtypeStruct((_N, _H), jnp.float32),
    jax.ShapeDtypeStruct((_N, _H), jnp.int32),
    jax.ShapeDtypeStruct((_N, _H), jnp.int32),
]


def _pack16(a, b):
    au = lax.bitcast_convert_type(a.astype(jnp.bfloat16),
                                  jnp.uint16).astype(jnp.int32)
    bu = lax.bitcast_convert_type(b.astype(jnp.bfloat16),
                                  jnp.uint16).astype(jnp.int32)
    return jnp.bitwise_or(au, lax.shift_left(bu, 16))


def _pack_role(p, c0):
    return jnp.concatenate(
        [_pack16(p[:, c0:c0 + 64], p[:, c0 + 64:c0 + 128]),
         _pack16(p[:, c0 + 128:c0 + 192], p[:, c0 + 192:c0 + 256])], axis=1)


def _split_proj(h, w512, h_ref, pd_ref, ps_ref):
    h_ref[...] = h
    p = jnp.dot(h, w512, preferred_element_type=jnp.float32)
    pd_ref[...] = _pack_role(p, 0)
    ps_ref[...] = _pack_role(p, 256)


def _stage0_body(atom_ref, wemb_ref, bemb_ref, w512_ref,
                 h_ref, pd_ref, ps_ref):
    a = atom_ref[...]
    h = jnp.dot(a, wemb_ref[...], preferred_element_type=jnp.float32) + bemb_ref[...]
    h = jnp.where(h > 0, h, jnp.exp(h) - 1.0)
    _split_proj(h, w512_ref[...], h_ref, pd_ref, ps_ref)


def _stage0(atom_feat, W_embed, b_embed, W512_0):
    return pl.pallas_call(
        _stage0_body,
        grid=(_N // _BM,),
        in_specs=[
            pl.BlockSpec((_BM, _H), lambda i: (i, 0)),
            pl.BlockSpec((_H, _H), lambda i: (0, 0)),
            pl.BlockSpec((1, _H), lambda i: (0, 0)),
            pl.BlockSpec((_H, 4 * _H), lambda i: (0, 0)),
        ],
        out_specs=_P_OUT_SPECS,
        out_shape=_P_OUT_SHAPE,
    )(atom_feat, W_embed, b_embed, W512_0)


def _stageL_body(h_ref, agg_ref, w512_ref, h_out_ref, pd_ref, ps_ref):
    h = h_ref[...] + agg_ref[0] + agg_ref[1]
    _split_proj(h, w512_ref[...], h_out_ref, pd_ref, ps_ref)


def _stageL(h, aggp, W512_l):
    return pl.pallas_call(
        _stageL_body,
        grid=(_N // _BM,),
        in_specs=[
            pl.BlockSpec((_BM, _H), lambda i: (i, 0)),
            pl.BlockSpec((_NC, _BM, _H), lambda i: (0, i, 0)),
            pl.BlockSpec((_H, 4 * _H), lambda i: (0, 0)),
        ],
        out_specs=_P_OUT_SPECS,
        out_shape=_P_OUT_SHAPE,
    )(h, aggp, W512_l)


def _bond_body(bfeat_ref, w_ref, b_ref, out_ref):
    p = (jnp.dot(bfeat_ref[...], w_ref[...],
                 preferred_element_type=jnp.float32) + b_ref[...])
    out_ref[...] = jnp.concatenate(
        [_pack16(p[:, :64], p[:, 64:128]),
         _pack16(p[:, 128:192], p[:, 192:])], axis=1)


def _bond(bond_feat, Wbond_l, bias_l):
    return pl.pallas_call(
        _bond_body,
        grid=(_E // _BME,),
        in_specs=[
            pl.BlockSpec((_BME, 16), lambda i: (i, 0)),
            pl.BlockSpec((16, 2 * _H), lambda i: (0, 0)),
            pl.BlockSpec((1, 2 * _H), lambda i: (0, 0)),
        ],
        out_specs=pl.BlockSpec((_BME, _H), lambda i: (i, 0)),
        out_shape=jax.ShapeDtypeStruct((_E, _H), jnp.int32),
    )(bond_feat, Wbond_l, bias_l)


def _pool_body(h_ref, agg_ref, gidx_ref, wout_ref, bout_ref, out_ref):
    h = h_ref[...] + agg_ref[0] + agg_ref[1]
    g = gidx_ref[...]
    iota = lax.broadcasted_iota(jnp.int32, (_G, _N), 0)
    onehot = (iota == g).astype(jnp.float32)
    sums = jnp.dot(onehot, h, preferred_element_type=jnp.float32)
    counts = jnp.sum(onehot, axis=1, keepdims=True)
    pooled = sums / jnp.maximum(counts, 1.0)
    out_ref[...] = (jnp.dot(pooled, wout_ref[...],
                            preferred_element_type=jnp.float32) + bout_ref[...])


def _pool(h, aggp, gidx, W_out, b_out):
    return pl.pallas_call(
        _pool_body,
        out_shape=jax.ShapeDtypeStruct((_G, _H), jnp.float32),
    )(h, aggp, gidx, W_out, b_out)


def kernel(atom_feat, bond_idx, graph_idx, bond_feat, W_embed, b_embed,
           Wf, bf, Ws, bs, W_out, b_out):
    src = bond_idx[0].astype(jnp.int32)
    dst = bond_idx[1].astype(jnp.int32)
    pk = (src | (dst << 14)).reshape(_NW, _NCHUNK, _CHUNK)
    gidx = graph_idx.astype(jnp.int32).reshape(1, _N)

    W512 = jnp.concatenate(
        [Wf[:, :_H, :], Ws[:, :_H, :], Wf[:, _H:2 * _H, :], Ws[:, _H:2 * _H, :]],
        axis=2)[:, :, _PERM512]
    Wbond = jnp.concatenate([Wf[:, 2 * _H:, :], Ws[:, 2 * _H:, :]],
                            axis=2)[:, :, _PERM256]
    bias = jnp.concatenate([bf, bs], axis=1)[:, _PERM256].reshape(_NL, 1, 2 * _H)

    h, pd, ps = _stage0(atom_feat, W_embed, b_embed.reshape(1, _H), W512[0])
    aggp = None
    for l in range(_NL):
        eb = _bond(bond_feat, Wbond[l], bias[l])
        aggp = _EDGE_PASS[0](pk, pd, ps, eb)
        if l + 1 < _NL:
            h, pd, ps = _stageL(h, aggp, W512[l + 1])
    return _pool(h, aggp, gidx, W_out, b_out.reshape(1, _H))


# speedup vs baseline: 1.0266x; 1.0056x over previous
import functools

import jax
import jax.numpy as jnp
import numpy as np
from jax import lax
from jax.experimental import pallas as pl
from jax.experimental.pallas import tpu as pltpu
from jax.experimental.pallas import tpu_sc as plsc

_N = 10000
_E = 320000
_H = 128
_NL = 4
_G = 64

_NC = 2
_NS = 16
_NW = _NC * _NS
_CHUNK = 80
_CA = 48
_CB = 32
_EPT = _E // _NW
_NCHUNK = _EPT // _CHUNK
_PKROWS = 64
_STRIPE = 640
_BLK = 80

_w = np.arange(64)
_NAT128 = np.concatenate([32 * (_w // 16) + (_w % 16),
                          32 * (_w // 16) + 16 + (_w % 16)]).astype(np.int32)
_PERM512 = np.concatenate([_NAT128 + 128 * b for b in range(4)])
_PERM256 = np.concatenate([_NAT128, _NAT128 + 128])

_LOG1P_C = (
    0.9996204346781663, -0.4866434251928923, 0.25462270211492277,
    -0.0747363009421317,
)


def _log1p01(t):
    p = jnp.full((16,), _LOG1P_C[-1], dtype=jnp.float32)
    for c in _LOG1P_C[-2::-1]:
        p = p * t + c
    return t * p


def _sigmoid(x):
    return 1.0 / (1.0 + jnp.exp(-x))


def _softplus(x):
    return jnp.maximum(x, 0.0) + _log1p01(jnp.exp(-jnp.abs(x)))


def _lo_f32(v):
    return lax.bitcast_convert_type(lax.shift_left(v, 16), jnp.float32)


def _hi_f32(v):
    return lax.bitcast_convert_type(jnp.bitwise_and(v, jnp.int32(-65536)),
                                    jnp.float32)


def _edge_body(layer, pk_hbm, pd_hbm, ps_hbm, eb_hbm, out_hbm,
               pk_t, sa_a, da_a, sa_b, da_b, d_sc,
               ad, as_, ae, bd, bs, be, msg_v, agg_sh, sema, semb):
    ecol = _H * layer
    cid = lax.axis_index("c")
    sid = lax.axis_index("s")
    wid = cid * _NS + sid

    zeros16 = jnp.zeros((16,), jnp.float32)

    def _zero_row(i, carry):
        for j in range(8):
            msg_v[i, pl.ds(16 * j, 16)] = zeros16
        return carry

    lax.fori_loop(0, _BLK, _zero_row, 0)
    nblk = jnp.where(sid == _NS - 1, (_N - (_NS - 1) * _STRIPE) // _BLK,
                     _STRIPE // _BLK)

    def _zero_blk(b, carry):
        pltpu.sync_copy(msg_v, agg_sh.at[pl.ds(sid * _STRIPE + b * _BLK, _BLK)])
        return carry

    lax.fori_loop(0, nblk, _zero_blk, 0)

    pltpu.sync_copy(pk_hbm.at[wid, pl.ds(0, _PKROWS)], pk_t)
    plsc.subcore_barrier()

    def _unpack(row, g0, g1, s_ref, d_ref):
        for g in range(g0, g1):
            v = pk_t[row, pl.ds(16 * g, 16)]
            s_ref[pl.ds(16 * (g - g0), 16)] = jnp.bitwise_and(
                v, jnp.int32(0x3FFF))
            d_ref[pl.ds(16 * (g - g0), 16)] = jnp.bitwise_and(
                lax.shift_right_logical(v, 14), jnp.int32(0x3FFF))

    def _act(gd, gs, ge, n, moff):
        def _body(i2, c2):
            for u in range(2):
                i = i2 * 2 + u
                for j in range(4):
                    w = 16 * j
                    vfd = gd[i, pl.ds(w, 16)]
                    vfs = gs[i, pl.ds(w, 16)]
                    vfe = ge[i, pl.ds(w, 16)]
                    vsd = gd[i, pl.ds(64 + w, 16)]
                    vss = gs[i, pl.ds(64 + w, 16)]
                    vse = ge[i, pl.ds(64 + w, 16)]
                    uf_lo = _lo_f32(vfd) + _lo_f32(vfs) + _lo_f32(vfe)
                    uf_hi = _hi_f32(vfd) + _hi_f32(vfs) + _hi_f32(vfe)
                    us_lo = _lo_f32(vsd) + _lo_f32(vss) + _lo_f32(vse)
                    us_hi = _hi_f32(vsd) + _hi_f32(vss) + _hi_f32(vse)
                    msg_v[moff + i, pl.ds(32 * j, 16)] = (
                        _sigmoid(uf_lo) * _softplus(us_lo))
                    msg_v[moff + i, pl.ds(32 * j + 16, 16)] = (
                        _sigmoid(uf_hi) * _softplus(us_hi))
            return c2

        lax.fori_loop(0, n // 2, _body, 0)

    def _ga(q):
        base = wid * _EPT + q * _CHUNK
        pltpu.async_copy(pd_hbm.at[da_a], ad, sema)
        pltpu.async_copy(ps_hbm.at[sa_a], as_, sema)
        pltpu.async_copy(eb_hbm.at[pl.ds(base, _CA), pl.ds(ecol, _H)], ae,
                         sema)

    def _gb(q):
        base = wid * _EPT + q * _CHUNK
        pltpu.async_copy(pd_hbm.at[da_b], bd, semb)
        pltpu.async_copy(ps_hbm.at[sa_b], bs, semb)
        pltpu.async_copy(eb_hbm.at[pl.ds(base + _CA, _CB), pl.ds(ecol, _H)],
                         be, semb)

    def _wait3(idx_ref, ref, sem):
        for _ in range(3):
            pltpu.make_async_copy(pd_hbm.at[idx_ref], ref, sem).wait()

    def _copy_idx(src_ref, dst_off, ngroups):
        for g in range(ngroups):
            d_sc[pl.ds(dst_off + 16 * g, 16)] = src_ref[pl.ds(16 * g, 16)]

    _unpack(0, 0, 3, sa_a, da_a)
    _unpack(0, 3, 5, sa_b, da_b)
    _ga(0)

    def _chunk(q, carry):
        _gb(q)

        @pl.when(q == _PKROWS - 1)
        def _():
            pltpu.sync_copy(pk_hbm.at[wid, pl.ds(_PKROWS, _NCHUNK - _PKROWS)],
                            pk_t.at[pl.ds(0, _NCHUNK - _PKROWS)])

        _wait3(da_a, ad, sema)
        _act(ad, as_, ae, _CA, 0)
        _copy_idx(da_a, 0, _CA // 16)
        row_n = jnp.where(q + 1 >= _PKROWS, q + 1 - _PKROWS, q + 1)

        @pl.when(q + 1 < _NCHUNK)
        def _():
            _unpack(row_n, 0, 3, sa_a, da_a)
            _ga(q + 1)

        _wait3(da_b, bd, semb)
        _act(bd, bs, be, _CB, _CA)
        _copy_idx(da_b, _CA, _CB // 16)
        pltpu.sync_copy(msg_v, agg_sh.at[d_sc], add=True)

        @pl.when(q + 1 < _NCHUNK)
        def _():
            _unpack(row_n, 3, 5, sa_b, da_b)

        return carry

    lax.fori_loop(0, _NCHUNK, _chunk, 0)
    plsc.subcore_barrier()

    def _out_blk(b, carry):
        r0 = sid * _STRIPE + b * _BLK
        pltpu.sync_copy(agg_sh.at[pl.ds(r0, _BLK)], msg_v)
        pltpu.sync_copy(msg_v, out_hbm.at[cid, pl.ds(r0, _BLK)])
        return carry

    lax.fori_loop(0, nblk, _out_blk, 0)


def _make_edge_pass(layer):
  return pl.kernel(
    functools.partial(_edge_body, layer),
    out_type=jax.ShapeDtypeStruct((_NC, _N, _H), jnp.float32),
    mesh=plsc.VectorSubcoreMesh(core_axis_name="c", subcore_axis_name="s"),
    scratch_types=[
        pltpu.VMEM((_PKROWS, _CHUNK), jnp.int32),
        pltpu.VMEM((_CA,), jnp.int32),
        pltpu.VMEM((_CA,), jnp.int32),
        pltpu.VMEM((_CB,), jnp.int32),
        pltpu.VMEM((_CB,), jnp.int32),
        pltpu.VMEM((_CHUNK,), jnp.int32),
        pltpu.VMEM((_CA, _H), jnp.int32),
        pltpu.VMEM((_CA, _H), jnp.int32),
        pltpu.VMEM((_CA, _H), jnp.int32),
        pltpu.VMEM((_CB, _H), jnp.int32),
        pltpu.VMEM((_CB, _H), jnp.int32),
        pltpu.VMEM((_CB, _H), jnp.int32),
        pltpu.VMEM((_CHUNK, _H), jnp.float32),
        pltpu.VMEM_SHARED((_N, _H), jnp.float32),
        pltpu.SemaphoreType.DMA,
        pltpu.SemaphoreType.DMA,
    ],
  )


_EDGE_PASS = [_make_edge_pass(l) for l in range(_NL)]


_BM = 1000
_BME = 4000

_P_OUT_SPECS = [
    pl.BlockSpec((_BM, _H), lambda i: (i, 0)),
    pl.BlockSpec((_BM, _H), lambda i: (i, 0)),
    pl.BlockSpec((_BM, _H), lambda i: (i, 0)),
]
_P_OUT_SHAPE = [
    jax.ShapeDtypeStruct((_N, _H), jnp.float32),
    jax.ShapeDtypeStruct((_N, _H), jnp.int32),
    jax.ShapeDtypeStruct((_N, _H), jnp.int32),
]


def _pack16(a, b):
    au = lax.bitcast_convert_type(a.astype(jnp.bfloat16),
                                  jnp.uint16).astype(jnp.int32)
    bu = lax.bitcast_convert_type(b.astype(jnp.bfloat16),
                                  jnp.uint16).astype(jnp.int32)
    return jnp.bitwise_or(au, lax.shift_left(bu, 16))


def _pack_role(p, c0):
    return jnp.concatenate(
        [_pack16(p[:, c0:c0 + 64], p[:, c0 + 64:c0 + 128]),
         _pack16(p[:, c0 + 128:c0 + 192], p[:, c0 + 192:c0 + 256])], axis=1)


def _split_proj(h, w512, h_ref, pd_ref, ps_ref):
    h_ref[...] = h
    p = jnp.dot(h, w512, preferred_element_type=jnp.float32)
    pd_ref[...] = _pack_role(p, 0)
    ps_ref[...] = _pack_role(p, 256)


def _stage0_body(atom_ref, wemb_ref, bemb_ref, w512_ref,
                 h_ref, pd_ref, ps_ref):
    a = atom_ref[...]
    h = jnp.dot(a, wemb_ref[...], preferred_element_type=jnp.float32) + bemb_ref[...]
    h = jnp.where(h > 0, h, jnp.exp(h) - 1.0)
    _split_proj(h, w512_ref[...], h_ref, pd_ref, ps_ref)


def _stage0(atom_feat, W_embed, b_embed, W512_0):
    return pl.pallas_call(
        _stage0_body,
        grid=(_N // _BM,),
        in_specs=[
            pl.BlockSpec((_BM, _H), lambda i: (i, 0)),
            pl.BlockSpec((_H, _H), lambda i: (0, 0)),
            pl.BlockSpec((1, _H), lambda i: (0, 0)),
            pl.BlockSpec((_H, 4 * _H), lambda i: (0, 0)),
        ],
        out_specs=_P_OUT_SPECS,
        out_shape=_P_OUT_SHAPE,
    )(atom_feat, W_embed, b_embed, W512_0)


def _stageL_body(h_ref, agg_ref, w512_ref, h_out_ref, pd_ref, ps_ref):
    h = h_ref[...] + agg_ref[0] + agg_ref[1]
    _split_proj(h, w512_ref[...], h_out_ref, pd_ref, ps_ref)


def _stageL(h, aggp, W512_l):
    return pl.pallas_call(
        _stageL_body,
        grid=(_N // _BM,),
        in_specs=[
            pl.BlockSpec((_BM, _H), lambda i: (i, 0)),
            pl.BlockSpec((_NC, _BM, _H), lambda i: (0, i, 0)),
            pl.BlockSpec((_H, 4 * _H), lambda i: (0, 0)),
        ],
        out_specs=_P_OUT_SPECS,
        out_shape=_P_OUT_SHAPE,
    )(h, aggp, W512_l)


def _bond_body(bfeat_ref, w_ref, b_ref, out_ref):
    p = (jnp.dot(bfeat_ref[...], w_ref[...],
                 preferred_element_type=jnp.float32) + b_ref[...])
    out_ref[...] = jnp.concatenate(
        [_pack16(p[:, :64], p[:, 64:128]),
         _pack16(p[:, 128:192], p[:, 192:])], axis=1)


def _bond(bond_feat, Wbond_l, bias_l):
    return pl.pallas_call(
        _bond_body,
        grid=(_E // _BME,),
        in_specs=[
            pl.BlockSpec((_BME, 16), lambda i: (i, 0)),
            pl.BlockSpec((16, 2 * _H), lambda i: (0, 0)),
            pl.BlockSpec((1, 2 * _H), lambda i: (0, 0)),
        ],
        out_specs=pl.BlockSpec((_BME, _H), lambda i: (i, 0)),
        out_shape=jax.ShapeDtypeStruct((_E, _H), jnp.int32),
    )(bond_feat, Wbond_l, bias_l)


def _pool_body(h_ref, agg_ref, gidx_ref, wout_ref, bout_ref, out_ref):
    h = h_ref[...] + agg_ref[0] + agg_ref[1]
    g = gidx_ref[...]
    iota = lax.broadcasted_iota(jnp.int32, (_G, _N), 0)
    onehot = (iota == g).astype(jnp.float32)
    sums = jnp.dot(onehot, h, preferred_element_type=jnp.float32)
    counts = jnp.sum(onehot, axis=1, keepdims=True)
    pooled = sums / jnp.maximum(counts, 1.0)
    out_ref[...] = (jnp.dot(pooled, wout_ref[...],
                            preferred_element_type=jnp.float32) + bout_ref[...])


def _pool(h, aggp, gidx, W_out, b_out):
    return pl.pallas_call(
        _pool_body,
        out_shape=jax.ShapeDtypeStruct((_G, _H), jnp.float32),
    )(h, aggp, gidx, W_out, b_out)


def kernel(atom_feat, bond_idx, graph_idx, bond_feat, W_embed, b_embed,
           Wf, bf, Ws, bs, W_out, b_out):
    src = bond_idx[0].astype(jnp.int32)
    dst = bond_idx[1].astype(jnp.int32)
    pk = (src | (dst << 14)).reshape(_NW, _NCHUNK, _CHUNK)
    gidx = graph_idx.astype(jnp.int32).reshape(1, _N)

    W512 = jnp.concatenate(
        [Wf[:, :_H, :], Ws[:, :_H, :], Wf[:, _H:2 * _H, :], Ws[:, _H:2 * _H, :]],
        axis=2)[:, :, _PERM512]
    Wbond = jnp.concatenate([Wf[:, 2 * _H:, :], Ws[:, 2 * _H:, :]],
                            axis=2)[:, :, _PERM256]
    bias = jnp.concatenate([bf, bs], axis=1)[:, _PERM256].reshape(_NL, 1, 2 * _H)

    h, pd, ps = _stage0(atom_feat, W_embed, b_embed.reshape(1, _H), W512[0])
    aggp = None
    for l in range(_NL):
        eb = _bond(bond_feat, Wbond[l], bias[l])
        aggp = _EDGE_PASS[0](pk, pd, ps, eb)
        if l + 1 < _NL:
            h, pd, ps = _stageL(h, aggp, W512[l + 1])
    return _pool(h, aggp, gidx, W_out, b_out.reshape(1, _H))
